# baseline (speedup 1.0000x reference)
import functools

import jax
import jax.numpy as jnp
from jax.experimental import pallas as pl
from jax.experimental.pallas import tpu as pltpu


def _ae_kernel(x_ref,
               wih1, whh1, b1,
               wih2, whh2, b2,
               wih3, whh3, b3,
               wih4, whh4, b4,
               wout, bout,
               out_ref, seq_ref, xg_ref, *, n_chunks, chunk):
    T, BB, F = x_ref.shape

    def gates(g, c, H):
        i = jax.nn.sigmoid(g[:, :H])
        f = jax.nn.sigmoid(g[:, H:2 * H])
        gc = jnp.tanh(g[:, 2 * H:3 * H])
        o = jax.nn.sigmoid(g[:, 3 * H:])
        c = f * c + i * gc
        return c, o * jnp.tanh(c)

    def lstm_chunked(read_chunk, wih_ref, whh_ref, b_ref):
        wih = wih_ref[...]
        whh = whh_ref[...]
        b = b_ref[...]
        H = whh.shape[0]
        G = 4 * H

        def chunk_body(ci, carry):
            xg = jnp.dot(read_chunk(ci), wih,
                         preferred_element_type=jnp.float32) + b
            xg_ref[:, :, :G] = xg.reshape(chunk, BB, G)

            def step(tl, carry2):
                h, c = carry2
                g = xg_ref[tl, :, :G] + jnp.dot(
                    h, whh, preferred_element_type=jnp.float32)
                c, hf = gates(g, c, H)
                h = hf.astype(jnp.bfloat16)
                seq_ref[ci * chunk + tl, :, :H] = h
                return h, c

            return jax.lax.fori_loop(0, chunk, step, carry)

        h, _ = jax.lax.fori_loop(
            0, n_chunks, chunk_body,
            (jnp.zeros((BB, H), jnp.bfloat16), jnp.zeros((BB, H), jnp.float32)))
        return h

    def lstm_repeated(xg_const, whh_ref):
        whh = whh_ref[...]
        H = whh.shape[0]

        def step(t, carry):
            h, c = carry
            g = xg_const + jnp.dot(h, whh, preferred_element_type=jnp.float32)
            c, hf = gates(g, c, H)
            h = hf.astype(jnp.bfloat16)
            seq_ref[t, :, :H] = h
            return h, c

        jax.lax.fori_loop(
            0, T, step,
            (jnp.zeros((BB, H), jnp.bfloat16), jnp.zeros((BB, H), jnp.float32)))

    def read_x(ci):
        return x_ref[pl.ds(ci * chunk, chunk), :, :].reshape(chunk * BB, F)

    lstm_chunked(read_x, wih1, whh1, b1)
    H1 = whh1.shape[0]

    def read_h1(ci):
        return seq_ref[pl.ds(ci * chunk, chunk), :, :H1].reshape(
            chunk * BB, H1)

    h_last = lstm_chunked(read_h1, wih2, whh2, b2)

    xg3 = jnp.dot(h_last, wih3[...],
                  preferred_element_type=jnp.float32) + b3[...]
    lstm_repeated(xg3, whh3)
    H3 = whh3.shape[0]

    def read_h3(ci):
        return seq_ref[pl.ds(ci * chunk, chunk), :, :H3].reshape(
            chunk * BB, H3)

    lstm_chunked(read_h3, wih4, whh4, b4)
    H4 = whh4.shape[0]

    y = jnp.dot(seq_ref[:, :, :H4].reshape(T * BB, H4), wout[...],
                preferred_element_type=jnp.float32) + bout[...]
    out_ref[...] = y.reshape(T, BB, F)


def _combine_gates(whh_g):
    _, H, _ = whh_g.shape
    return jnp.transpose(whh_g, (1, 0, 2)).reshape(H, 4 * H)


@jax.jit
def kernel(data, p00, p01, p02, p03, p04, p05, p06, p07, p08, p09, p10,
           p11, p12, p13):
    B, T, F = data.shape
    BB = 128 if B % 128 == 0 else B
    chunk = 16 if T % 16 == 0 else T
    n_chunks = T // chunk

    bf = jnp.bfloat16
    params = (p00.astype(bf), _combine_gates(p01).astype(bf), p02,
              p03.astype(bf), _combine_gates(p04).astype(bf), p05,
              p06.astype(bf), _combine_gates(p07).astype(bf), p08,
              p09.astype(bf), _combine_gates(p10).astype(bf), p11,
              p12.astype(bf), p13)

    h_max = max(p01.shape[2], p04.shape[2], p07.shape[2], p10.shape[2])
    x_tbf = jnp.transpose(data.astype(bf), (1, 0, 2))

    def whole(arr):
        return pl.BlockSpec(arr.shape, lambda b, _nd=arr.ndim: (0,) * _nd)

    out = pl.pallas_call(
        functools.partial(_ae_kernel, n_chunks=n_chunks, chunk=chunk),
        out_shape=jax.ShapeDtypeStruct((T, B, F), jnp.float32),
        grid=(B // BB,),
        in_specs=([pl.BlockSpec((T, BB, F), lambda b: (0, b, 0))]
                  + [whole(w) for w in params]),
        out_specs=pl.BlockSpec((T, BB, F), lambda b: (0, b, 0)),
        scratch_shapes=[pltpu.VMEM((T, BB, h_max), jnp.bfloat16),
                        pltpu.VMEM((chunk, BB, 4 * h_max), jnp.float32)],
        compiler_params=pltpu.CompilerParams(
            dimension_semantics=("parallel",),
            vmem_limit_bytes=64 * 1024 * 1024),
    )(x_tbf, *params)
    return jnp.transpose(out, (1, 0, 2))


# speedup vs baseline: 1.2558x; 1.2558x over previous
import functools

import jax
import jax.numpy as jnp
from jax.experimental import pallas as pl
from jax.experimental.pallas import tpu as pltpu


def _ae_kernel(x_ref,
               wih1, whh1, b1,
               wih2, whh2, b2,
               wih3, whh3, b3,
               wih4, whh4, b4,
               wout, bout,
               out_ref, seq_ref, xg_ref, *, n_chunks, chunk):
    T, BB, F = x_ref.shape
    HF = BB // 2

    def gates(g, c, H):
        i = jax.nn.sigmoid(g[:, :H])
        f = jax.nn.sigmoid(g[:, H:2 * H])
        gc = jnp.tanh(g[:, 2 * H:3 * H])
        o = jax.nn.sigmoid(g[:, 3 * H:])
        c = f * c + i * gc
        return c, o * jnp.tanh(c)

    def zstate(H):
        return (jnp.zeros((HF, H), jnp.bfloat16),
                jnp.zeros((HF, H), jnp.float32),
                jnp.zeros((HF, H), jnp.bfloat16),
                jnp.zeros((HF, H), jnp.float32))

    def lstm_chunked(read_chunk, wih_ref, whh_ref, b_ref):
        wih = wih_ref[...]
        whh = whh_ref[...]
        b = b_ref[...]
        H = whh.shape[0]
        G = 4 * H

        def chunk_body(ci, carry):
            xg = jnp.dot(read_chunk(ci), wih,
                         preferred_element_type=jnp.float32) + b
            xg_ref[:, :, :G] = xg.reshape(chunk, BB, G)

            def step(tl, carry2):
                ha, ca, hb, cb = carry2
                ga = xg_ref[tl, :HF, :G] + jnp.dot(
                    ha, whh, preferred_element_type=jnp.float32)
                gb = xg_ref[tl, HF:, :G] + jnp.dot(
                    hb, whh, preferred_element_type=jnp.float32)
                ca, hfa = gates(ga, ca, H)
                cb, hfb = gates(gb, cb, H)
                ha = hfa.astype(jnp.bfloat16)
                hb = hfb.astype(jnp.bfloat16)
                seq_ref[ci * chunk + tl, :HF, :H] = ha
                seq_ref[ci * chunk + tl, HF:, :H] = hb
                return ha, ca, hb, cb

            return jax.lax.fori_loop(0, chunk, step, carry)

        ha, _, hb, _ = jax.lax.fori_loop(0, n_chunks, chunk_body, zstate(H))
        return jnp.concatenate([ha, hb], axis=0)

    def lstm_repeated(xg_const, whh_ref):
        whh = whh_ref[...]
        H = whh.shape[0]
        xga = xg_const[:HF]
        xgb = xg_const[HF:]

        def step(t, carry):
            ha, ca, hb, cb = carry
            ga = xga + jnp.dot(ha, whh, preferred_element_type=jnp.float32)
            gb = xgb + jnp.dot(hb, whh, preferred_element_type=jnp.float32)
            ca, hfa = gates(ga, ca, H)
            cb, hfb = gates(gb, cb, H)
            ha = hfa.astype(jnp.bfloat16)
            hb = hfb.astype(jnp.bfloat16)
            seq_ref[t, :HF, :H] = ha
            seq_ref[t, HF:, :H] = hb
            return ha, ca, hb, cb

        jax.lax.fori_loop(0, T, step, zstate(H))

    def read_x(ci):
        return x_ref[pl.ds(ci * chunk, chunk), :, :].reshape(chunk * BB, F)

    lstm_chunked(read_x, wih1, whh1, b1)
    H1 = whh1.shape[0]

    def read_h1(ci):
        return seq_ref[pl.ds(ci * chunk, chunk), :, :H1].reshape(
            chunk * BB, H1)

    h_last = lstm_chunked(read_h1, wih2, whh2, b2)

    xg3 = jnp.dot(h_last, wih3[...],
                  preferred_element_type=jnp.float32) + b3[...]
    lstm_repeated(xg3, whh3)
    H3 = whh3.shape[0]

    def read_h3(ci):
        return seq_ref[pl.ds(ci * chunk, chunk), :, :H3].reshape(
            chunk * BB, H3)

    lstm_chunked(read_h3, wih4, whh4, b4)
    H4 = whh4.shape[0]

    y = jnp.dot(seq_ref[:, :, :H4].reshape(T * BB, H4), wout[...],
                preferred_element_type=jnp.float32) + bout[...]
    out_ref[...] = y.reshape(T, BB, F)


def _combine_gates(whh_g):
    _, H, _ = whh_g.shape
    return jnp.transpose(whh_g, (1, 0, 2)).reshape(H, 4 * H)


@jax.jit
def kernel(data, p00, p01, p02, p03, p04, p05, p06, p07, p08, p09, p10,
           p11, p12, p13):
    B, T, F = data.shape
    BB = 256 if B % 256 == 0 else B
    chunk = 16 if T % 16 == 0 else T
    n_chunks = T // chunk

    bf = jnp.bfloat16
    params = (p00.astype(bf), _combine_gates(p01).astype(bf), p02,
              p03.astype(bf), _combine_gates(p04).astype(bf), p05,
              p06.astype(bf), _combine_gates(p07).astype(bf), p08,
              p09.astype(bf), _combine_gates(p10).astype(bf), p11,
              p12.astype(bf), p13)

    h_max = max(p01.shape[2], p04.shape[2], p07.shape[2], p10.shape[2])
    x_tbf = jnp.transpose(data.astype(bf), (1, 0, 2))

    def whole(arr):
        return pl.BlockSpec(arr.shape, lambda b, _nd=arr.ndim: (0,) * _nd)

    out = pl.pallas_call(
        functools.partial(_ae_kernel, n_chunks=n_chunks, chunk=chunk),
        out_shape=jax.ShapeDtypeStruct((T, B, F), jnp.float32),
        grid=(B // BB,),
        in_specs=([pl.BlockSpec((T, BB, F), lambda b: (0, b, 0))]
                  + [whole(w) for w in params]),
        out_specs=pl.BlockSpec((T, BB, F), lambda b: (0, b, 0)),
        scratch_shapes=[pltpu.VMEM((T, BB, h_max), jnp.bfloat16),
                        pltpu.VMEM((chunk, BB, 4 * h_max), jnp.float32)],
        compiler_params=pltpu.CompilerParams(
            dimension_semantics=("parallel",),
            vmem_limit_bytes=64 * 1024 * 1024),
    )(x_tbf, *params)
    return jnp.transpose(out, (1, 0, 2))


# speedup vs baseline: 1.2582x; 1.0019x over previous
import functools

import jax
import jax.numpy as jnp
from jax.experimental import pallas as pl
from jax.experimental.pallas import tpu as pltpu


def _ae_kernel(x_ref,
               wih1, whh1, b1,
               wih2, whh2, b2,
               wih3, whh3, b3,
               wih4, whh4, b4,
               wout, bout,
               out_ref, seq_ref, xg_ref, *, n_chunks, chunk):
    T, BB, F = x_ref.shape
    HF = BB // 2

    def gates(g, c, H):
        i = jax.nn.sigmoid(g[:, :H])
        f = jax.nn.sigmoid(g[:, H:2 * H])
        gc = jnp.tanh(g[:, 2 * H:3 * H])
        o = jax.nn.sigmoid(g[:, 3 * H:])
        c = f * c + i * gc
        return c, o * jnp.tanh(c)

    def zstate(H):
        return (jnp.zeros((HF, H), jnp.bfloat16),
                jnp.zeros((HF, H), jnp.float32),
                jnp.zeros((HF, H), jnp.bfloat16),
                jnp.zeros((HF, H), jnp.float32))

    def lstm_chunked(read_chunk, wih_ref, whh_ref, b_ref):
        wih = wih_ref[...]
        whh = whh_ref[...]
        b = b_ref[...]
        H = whh.shape[0]
        G = 4 * H

        def chunk_body(ci, carry):
            xg = jnp.dot(read_chunk(ci), wih,
                         preferred_element_type=jnp.float32) + b
            xg_ref[:, :, :G] = xg.reshape(chunk, BB, G).astype(jnp.bfloat16)

            def step(tl, carry2):
                ha, ca, hb, cb = carry2
                ga = xg_ref[tl, :HF, :G].astype(jnp.float32) + jnp.dot(
                    ha, whh, preferred_element_type=jnp.float32)
                gb = xg_ref[tl, HF:, :G].astype(jnp.float32) + jnp.dot(
                    hb, whh, preferred_element_type=jnp.float32)
                ca, hfa = gates(ga, ca, H)
                cb, hfb = gates(gb, cb, H)
                ha = hfa.astype(jnp.bfloat16)
                hb = hfb.astype(jnp.bfloat16)
                seq_ref[ci * chunk + tl, :HF, :H] = ha
                seq_ref[ci * chunk + tl, HF:, :H] = hb
                return ha, ca, hb, cb

            return jax.lax.fori_loop(0, chunk, step, carry)

        ha, _, hb, _ = jax.lax.fori_loop(0, n_chunks, chunk_body, zstate(H))
        return jnp.concatenate([ha, hb], axis=0)

    def lstm_repeated(xg_const, whh_ref):
        whh = whh_ref[...]
        H = whh.shape[0]
        xga = xg_const[:HF]
        xgb = xg_const[HF:]

        def step(t, carry):
            ha, ca, hb, cb = carry
            ga = xga + jnp.dot(ha, whh, preferred_element_type=jnp.float32)
            gb = xgb + jnp.dot(hb, whh, preferred_element_type=jnp.float32)
            ca, hfa = gates(ga, ca, H)
            cb, hfb = gates(gb, cb, H)
            ha = hfa.astype(jnp.bfloat16)
            hb = hfb.astype(jnp.bfloat16)
            seq_ref[t, :HF, :H] = ha
            seq_ref[t, HF:, :H] = hb
            return ha, ca, hb, cb

        jax.lax.fori_loop(0, T, step, zstate(H))

    def read_x(ci):
        return x_ref[pl.ds(ci * chunk, chunk), :, :].reshape(chunk * BB, F)

    lstm_chunked(read_x, wih1, whh1, b1)
    H1 = whh1.shape[0]

    def read_h1(ci):
        return seq_ref[pl.ds(ci * chunk, chunk), :, :H1].reshape(
            chunk * BB, H1)

    h_last = lstm_chunked(read_h1, wih2, whh2, b2)

    xg3 = jnp.dot(h_last, wih3[...],
                  preferred_element_type=jnp.float32) + b3[...]
    lstm_repeated(xg3, whh3)
    H3 = whh3.shape[0]

    def read_h3(ci):
        return seq_ref[pl.ds(ci * chunk, chunk), :, :H3].reshape(
            chunk * BB, H3)

    lstm_chunked(read_h3, wih4, whh4, b4)
    H4 = whh4.shape[0]

    y = jnp.dot(seq_ref[:, :, :H4].reshape(T * BB, H4), wout[...],
                preferred_element_type=jnp.float32) + bout[...]
    out_ref[...] = y.reshape(T, BB, F)


def _combine_gates(whh_g):
    _, H, _ = whh_g.shape
    return jnp.transpose(whh_g, (1, 0, 2)).reshape(H, 4 * H)


@jax.jit
def kernel(data, p00, p01, p02, p03, p04, p05, p06, p07, p08, p09, p10,
           p11, p12, p13):
    B, T, F = data.shape
    BB = 256 if B % 256 == 0 else B
    chunk = 16 if T % 16 == 0 else T
    n_chunks = T // chunk

    bf = jnp.bfloat16
    params = (p00.astype(bf), _combine_gates(p01).astype(bf), p02,
              p03.astype(bf), _combine_gates(p04).astype(bf), p05,
              p06.astype(bf), _combine_gates(p07).astype(bf), p08,
              p09.astype(bf), _combine_gates(p10).astype(bf), p11,
              p12.astype(bf), p13)

    h_max = max(p01.shape[2], p04.shape[2], p07.shape[2], p10.shape[2])
    x_tbf = jnp.transpose(data.astype(bf), (1, 0, 2))

    def whole(arr):
        return pl.BlockSpec(arr.shape, lambda b, _nd=arr.ndim: (0,) * _nd)

    out = pl.pallas_call(
        functools.partial(_ae_kernel, n_chunks=n_chunks, chunk=chunk),
        out_shape=jax.ShapeDtypeStruct((T, B, F), jnp.float32),
        grid=(B // BB,),
        in_specs=([pl.BlockSpec((T, BB, F), lambda b: (0, b, 0))]
                  + [whole(w) for w in params]),
        out_specs=pl.BlockSpec((T, BB, F), lambda b: (0, b, 0)),
        scratch_shapes=[pltpu.VMEM((T, BB, h_max), jnp.bfloat16),
                        pltpu.VMEM((chunk, BB, 4 * h_max), jnp.bfloat16)],
        compiler_params=pltpu.CompilerParams(
            dimension_semantics=("parallel",),
            vmem_limit_bytes=64 * 1024 * 1024),
    )(x_tbf, *params)
    return jnp.transpose(out, (1, 0, 2))
